# baseline (speedup 1.0000x reference)
import functools

import jax
import jax.numpy as jnp
from jax import lax
from jax.experimental import pallas as pl
from jax.experimental.pallas import tpu as pltpu
from jax.experimental.pallas import tpu_sc as plsc

B = 16384
D = 64
NC = 2
NS = 16
NW = NC * NS
BPW = B // NW
WAVE = 128
NWAVE = BPW // WAVE
NBROW = 7813


def _body(u_idx_hbm, i_idx_hbm, ut2_hbm, ubp_hbm, it2_hbm, ibp_hbm,
          out_hbm,
          iv_u, iv_i, r2u, r2i, r3u, r3i,
          gbu, gbi, gbub, gbib, out_v, sem):
    wid = lax.axis_index("s") * NC + lax.axis_index("c")
    base = wid * BPW

    pltpu.sync_copy(u_idx_hbm.at[pl.ds(base, BPW)], iv_u)
    pltpu.sync_copy(i_idx_hbm.at[pl.ds(base, BPW)], iv_i)

    for k in range(BPW // 16):
        w, off = k // (WAVE // 16), (k % (WAVE // 16)) * 16
        sl = pl.ds(k * 16, 16)
        dsl = pl.ds(off, 16)
        u16 = iv_u[sl]
        i16 = iv_i[sl]
        r2u[w, dsl] = lax.shift_right_logical(u16, 1)
        r2i[w, dsl] = lax.shift_right_logical(i16, 1)
        r3u[w, dsl] = lax.shift_right_logical(u16, 7)
        r3i[w, dsl] = lax.shift_right_logical(i16, 7)

    lanes = lax.iota(jnp.int32, 16)

    for w in range(NWAVE):
        cps = (
            pltpu.make_async_copy(ut2_hbm.at[r2u.at[w]], gbu, sem),
            pltpu.make_async_copy(it2_hbm.at[r2i.at[w]], gbi, sem),
            pltpu.make_async_copy(ubp_hbm.at[r3u.at[w]], gbub, sem),
            pltpu.make_async_copy(ibp_hbm.at[r3i.at[w]], gbib, sem),
        )
        for cp in cps:
            cp.start()
        for cp in cps:
            cp.wait()

        def grp(g, _, w=w):
            sl = pl.ds(w * WAVE + g * 16, 16)
            lsl = pl.ds(g * 16, 16)
            u16 = iv_u[sl]
            i16 = iv_i[sl]
            rr16 = g * 16 + lanes
            offu = (u16 & 1) * D
            offi = (i16 & 1) * D

            def col(c, acc):
                vu = plsc.load_gather(gbu, [rr16, offu + c])
                vi = plsc.load_gather(gbi, [rr16, offi + c])
                return acc + vu * vi

            acc0 = (plsc.load_gather(gbub, [rr16, u16 & 127])
                    + plsc.load_gather(gbib, [rr16, i16 & 127]))
            x = lax.fori_loop(0, D, col, acc0)
            out_v[sl] = 1.0 / (1.0 + jnp.exp(-x))
            return 0

        lax.fori_loop(0, WAVE // 16, grp, 0)

    pltpu.sync_copy(out_v, out_hbm.at[pl.ds(base, BPW)])


@functools.partial(jax.jit, static_argnames=())
def _run(u_idx, i_idx, ut2, ubp, it2, ibp):
    mesh = plsc.VectorSubcoreMesh(core_axis_name="c", subcore_axis_name="s",
                                  num_cores=NC, num_subcores=NS)
    f = pl.kernel(
        _body,
        out_type=jax.ShapeDtypeStruct((B,), jnp.float32),
        mesh=mesh,
        compiler_params=pltpu.CompilerParams(needs_layout_passes=False,
                                             use_tc_tiling_on_sc=True),
        scratch_types=[
            pltpu.VMEM((BPW,), jnp.int32),
            pltpu.VMEM((BPW,), jnp.int32),
            pltpu.VMEM((NWAVE, WAVE), jnp.int32),
            pltpu.VMEM((NWAVE, WAVE), jnp.int32),
            pltpu.VMEM((NWAVE, WAVE), jnp.int32),
            pltpu.VMEM((NWAVE, WAVE), jnp.int32),
            pltpu.VMEM((WAVE, 128), jnp.float32),
            pltpu.VMEM((WAVE, 128), jnp.float32),
            pltpu.VMEM((WAVE, 128), jnp.float32),
            pltpu.VMEM((WAVE, 128), jnp.float32),
            pltpu.VMEM((BPW,), jnp.float32),
            pltpu.SemaphoreType.DMA,
        ],
    )
    return f(u_idx, i_idx, ut2, ubp, it2, ibp)


NCOL = 7813
CPW = (NCOL + NW - 1) // NW
NBUF = 4


def _conv_body(src_hbm, out_hbm, *scratch):
    ib = scratch[0:NBUF]
    ob = scratch[NBUF:2 * NBUF]
    si = scratch[2 * NBUF:3 * NBUF]
    so = scratch[3 * NBUF:4 * NBUF]
    wid = lax.axis_index("s") * NC + lax.axis_index("c")
    t0 = wid * CPW
    nch = jnp.minimum(CPW, NCOL - t0)
    lanes = lax.iota(jnp.int32, 16)

    def in_copy(t_local, b):
        tg = pl.multiple_of((t0 + t_local) * 128, 128)
        return pltpu.make_async_copy(src_hbm.at[:, pl.ds(tg, 128)],
                                     ib[b], si[b])

    def out_copy(t_local, b):
        rg = pl.multiple_of((t0 + t_local) * 64, 64)
        return pltpu.make_async_copy(ob[b], out_hbm.at[pl.ds(rg, 64), :],
                                     so[b])

    def step(m, _):
        for b in range(NBUF):
            k = NBUF * m + b

            @pl.when(k < nch)
            def _(k=k, b=b):
                in_copy(k, b).start()

            c = k - (NBUF - 1)
            bc = (b + 1) % NBUF

            @pl.when((c >= 0) & (c < nch))
            def _(c=c, bc=bc):
                in_copy(c, bc).wait()

                @pl.when(c >= NBUF)
                def _():
                    out_copy(c - NBUF, bc).wait()

                def rows(r2, _):
                    for rr in range(2):
                        r = 2 * r2 + rr
                        le = jnp.full((16,), 2 * r, jnp.int32)
                        lo = jnp.full((16,), 2 * r + 1, jnp.int32)
                        for k4 in range(4):
                            fv = k4 * 16 + lanes
                            ob[bc][r, pl.ds(k4 * 16, 16)] = (
                                plsc.load_gather(ib[bc], [fv, le]))
                            ob[bc][r, pl.ds(64 + k4 * 16, 16)] = (
                                plsc.load_gather(ib[bc], [fv, lo]))
                    return 0

                lax.fori_loop(0, 32, rows, 0)
                out_copy(c, bc).start()

        return 0

    lax.fori_loop(0, (CPW + 2 * NBUF) // NBUF + 1, step, 0)

    for b in range(NBUF):
        cb = ((nch - 1 - b) // NBUF) * NBUF + b

        @pl.when((cb >= 0) & (cb < nch))
        def _(cb=cb, b=b):
            out_copy(cb, b).wait()


@functools.partial(jax.jit, static_argnames=())
def _convert(src):
    mesh = plsc.VectorSubcoreMesh(core_axis_name="c", subcore_axis_name="s",
                                  num_cores=NC, num_subcores=NS)
    f = pl.kernel(
        _conv_body,
        out_type=jax.ShapeDtypeStruct((NCOL * 64, 128), jnp.float32),
        mesh=mesh,
        compiler_params=pltpu.CompilerParams(needs_layout_passes=False,
                                             use_tc_tiling_on_sc=True),
        scratch_types=(
            [pltpu.VMEM((D, 128), jnp.float32)] * NBUF
            + [pltpu.VMEM((D, 128), jnp.float32)] * NBUF
            + [pltpu.SemaphoreType.DMA] * (2 * NBUF)
        ),
    )
    return f(src)


def kernel(inputs, user_embedding, user_bias, item_embedding, item_bias):
    u_idx = inputs[:, 0]
    i_idx = inputs[:, 1]
    ut2 = _convert(user_embedding.T)
    it2 = _convert(item_embedding.T)
    ubp = jnp.pad(user_bias[:, 0], (0, NBROW * 128 - user_bias.shape[0])
                  ).reshape(NBROW, 128)
    ibp = jnp.pad(item_bias[:, 0], (0, NBROW * 128 - item_bias.shape[0])
                  ).reshape(NBROW, 128)
    out = _run(u_idx, i_idx, ut2, ubp, it2, ibp)
    return out[:, None]


# speedup vs baseline: 2.7500x; 2.7500x over previous
import functools

import jax
import jax.numpy as jnp
from jax import lax
from jax.experimental import pallas as pl
from jax.experimental.pallas import tpu as pltpu
from jax.experimental.pallas import tpu_sc as plsc

B = 16384
D = 64
NC = 2
NS = 16
NW = NC * NS
BPW = B // NW
CHUNK = 128
NCHUNK = BPW // CHUNK


def _body(u_idx_hbm, i_idx_hbm, user_emb_hbm, ub_hbm, item_emb_hbm, ib_hbm,
          out_hbm,
          u_idx_v, i_idx_v, u_rows, i_rows, ub_v, ib_v, p_v, out_v, sem):
    wid = lax.axis_index("s") * NC + lax.axis_index("c")
    base = wid * BPW

    for j in range(NCHUNK):
        pltpu.sync_copy(u_idx_hbm.at[pl.ds(base + j * CHUNK, CHUNK)], u_idx_v.at[j])
        pltpu.sync_copy(i_idx_hbm.at[pl.ds(base + j * CHUNK, CHUNK)], i_idx_v.at[j])

    copies = []
    for j in range(NCHUNK):
        sl = pl.ds(j * CHUNK, CHUNK)
        copies.append(pltpu.async_copy(user_emb_hbm.at[u_idx_v.at[j]], u_rows.at[sl], sem))
        copies.append(pltpu.async_copy(item_emb_hbm.at[i_idx_v.at[j]], i_rows.at[sl], sem))
        copies.append(pltpu.async_copy(ub_hbm.at[u_idx_v.at[j]], ub_v.at[sl], sem))
        copies.append(pltpu.async_copy(ib_hbm.at[i_idx_v.at[j]], ib_v.at[sl], sem))
    for c in copies:
        c.wait()

    def row(r, _):
        p = u_rows[r, pl.ds(0, 16)] * i_rows[r, pl.ds(0, 16)]
        for c0 in range(16, D, 16):
            p = p + u_rows[r, pl.ds(c0, 16)] * i_rows[r, pl.ds(c0, 16)]
        p_v[pl.ds(r * 16, 16)] = p
        return 0

    lax.fori_loop(0, BPW, row, 0)

    lanes = lax.iota(jnp.int32, 16)

    def group(g, _):
        sl = pl.ds(g * 16, 16)
        base_idx = (g * 16 + lanes) * 16
        x = plsc.load_gather(p_v, [base_idx])
        for c in range(1, 16):
            x = x + plsc.load_gather(p_v, [base_idx + c])
        x = x + ub_v[sl] + ib_v[sl]
        out_v[sl] = 1.0 / (1.0 + jnp.exp(-x))
        return 0

    lax.fori_loop(0, BPW // 16, group, 0)

    pltpu.sync_copy(out_v, out_hbm.at[pl.ds(base, BPW)])


@functools.partial(jax.jit, static_argnames=())
def _run(u_idx, i_idx, user_emb, ub, item_emb, ib):
    mesh = plsc.VectorSubcoreMesh(core_axis_name="c", subcore_axis_name="s",
                                  num_cores=NC, num_subcores=NS)
    f = pl.kernel(
        _body,
        out_type=jax.ShapeDtypeStruct((B,), jnp.float32),
        mesh=mesh,
        compiler_params=pltpu.CompilerParams(needs_layout_passes=False,
                                             use_tc_tiling_on_sc=False),
        scratch_types=[
            pltpu.VMEM((NCHUNK, CHUNK), jnp.int32),
            pltpu.VMEM((NCHUNK, CHUNK), jnp.int32),
            pltpu.VMEM((BPW, D), jnp.float32),
            pltpu.VMEM((BPW, D), jnp.float32),
            pltpu.VMEM((BPW,), jnp.float32),
            pltpu.VMEM((BPW,), jnp.float32),
            pltpu.VMEM((BPW * 16,), jnp.float32),
            pltpu.VMEM((BPW,), jnp.float32),
            pltpu.SemaphoreType.DMA,
        ],
    )
    return f(u_idx, i_idx, user_emb, ub, item_emb, ib)


def kernel(inputs, user_embedding, user_bias, item_embedding, item_bias):
    u_idx = inputs[:, 0]
    i_idx = inputs[:, 1]
    out = _run(u_idx, i_idx, user_embedding, user_bias[:, 0],
               item_embedding, item_bias[:, 0])
    return out[:, None]
